# Initial kernel scaffold; baseline (speedup 1.0000x reference)
import jax
import jax.numpy as jnp
from jax.experimental import pallas as pl


def kernel(msa, pair, xyz, state, idx, params):
    raise NotImplementedError("write your pallas kernel here")


# speedup vs baseline: 2.6367x; 2.6367x over previous
import functools

import jax
import jax.numpy as jnp
from jax.experimental import pallas as pl
from jax.experimental.pallas import tpu as pltpu

L = 384
TOP_K = 128
BI = 32
EXP_BITS_INF = 0x7F800000


def _ln(x, g, b, eps=1e-5):
    g = g.reshape((1,) * (x.ndim - 1) + (-1,))
    b = b.reshape((1,) * (x.ndim - 1) + (-1,))
    m = jnp.mean(x, -1, keepdims=True)
    xc = x - m
    v = jnp.mean(xc * xc, -1, keepdims=True)
    return xc / jnp.sqrt(v + eps) * g + b


def _dot(x, w):
    return jnp.dot(x, w, preferred_element_type=jnp.float32)


def _body(msa_ref, state_ref, xyz_ref, idx_ref, pair_ref, w_ref,
          oxyz_ref, ostate_ref, oalpha_ref):
    i = pl.program_id(0)
    r0 = i * BI
    w = {k: v[:] for k, v in w_ref.items()}

    lnm = _ln(msa_ref[:], w['g_msa'], w['b_msa'])
    lnst = _ln(state_ref[:], w['g_state'], w['b_state'])
    node = _ln(_dot(lnm, w['Wx_m']) + _dot(lnst, w['Wx_s']) + w['bx'],
               w['g_node'], w['b_node'])

    xyz_all = xyz_ref[:]
    ca_all = xyz_all[:, 1, :]
    xyz_blk = xyz_ref[pl.ds(r0, BI), :, :]
    ca_blk = xyz_blk[:, 1, :]
    diff = ca_blk[:, None, :] - ca_all[None, :, :]
    D3 = jnp.sqrt(jnp.sum(diff * diff, -1, keepdims=True) + 1e-8)

    bits = jax.lax.bitcast_convert_type(D3, jnp.int32)
    lo0 = jnp.zeros((BI, 1, 1), jnp.int32)
    hi0 = jnp.full((BI, 1, 1), EXP_BITS_INF, jnp.int32)

    def bs_step(_, lohi):
        lo, hi = lohi
        mid = lo + ((hi - lo) >> 1)
        cnt = jnp.sum((bits <= mid).astype(jnp.float32), axis=1, keepdims=True)
        pred = cnt >= float(TOP_K)
        return jnp.where(pred, lo, mid), jnp.where(pred, mid, hi)

    _, T = jax.lax.fori_loop(0, 31, bs_step, (lo0, hi0))
    lt = bits < T
    eqm = bits == T
    n_lt = jnp.sum(lt.astype(jnp.float32), axis=1, keepdims=True)
    coli = jax.lax.broadcasted_iota(jnp.int32, (1, L, 1), 1)

    jlo0 = jnp.full((BI, 1, 1), -1, jnp.int32)
    jhi0 = jnp.full((BI, 1, 1), L - 1, jnp.int32)

    def js_step(_, lohi):
        lo, hi = lohi
        mid = lo + ((hi - lo) >> 1)
        cnt = n_lt + jnp.sum((eqm & (coli <= mid)).astype(jnp.float32),
                             axis=1, keepdims=True)
        pred = cnt >= float(TOP_K)
        return jnp.where(pred, lo, mid), jnp.where(pred, mid, hi)

    _, J = jax.lax.fori_loop(0, 9, js_step, (jlo0, jhi0))
    mask3 = (lt | (eqm & (coli <= J))).astype(jnp.float32)

    prn = _ln(pair_ref[:], w['g_pair'], w['b_pair'])
    pr1 = _ln(_dot(prn.reshape(BI * L, 128), w['W_e1']) + w['b_e1'],
              w['g_e1n'], w['b_e1n'])
    mu3 = 2.0 + jax.lax.broadcasted_iota(jnp.int32, (1, 1, 36), 2).astype(
        jnp.float32) * (20.0 / 35.0)
    rbf = jnp.exp(-jnp.square(D3 - mu3)).reshape(BI * L, 36)
    idx_all3 = idx_ref[:][None, :, :]
    idx_blk3 = idx_ref[pl.ds(r0, BI), :][:, :, None]
    nb = jnp.log(jnp.abs(idx_blk3 - idx_all3) + 1.0).reshape(BI * L, 1)
    e2 = (_dot(pr1, w['W_e2_pr']) + _dot(rbf, w['W_e2_rbf'])
          + nb * w['w_e2_nb'] + w['b_e2'])
    edge = _ln(e2, w['g_e2n'], w['b_e2n'])

    lnm_b = _ln(msa_ref[pl.ds(r0, BI), :], w['g_msa'], w['b_msa'])
    lnst_b = _ln(state_ref[pl.ds(r0, BI), :], w['g_state'], w['b_state'])
    node_blk = _ln(_dot(lnm_b, w['Wx_m']) + _dot(lnst_b, w['Wx_s']) + w['bx'],
                   w['g_node'], w['b_node'])
    t_self = _dot(node_blk, w['Wm_self'])
    t_nbr = _dot(node, w['Wm_nbr'])
    t_edge = _dot(edge, w['Wm_edge']).reshape(BI, L, 32)
    h = jax.nn.relu(t_edge + t_nbr[None, :, :] + t_self[:, None, :]
                    + w['bm'].reshape(1, 1, 32))

    aggm = jnp.sum(mask3 * h, axis=1) / float(TOP_K)
    state_out = _dot(node_blk, w['Wl0_node']) + _dot(aggm, w['Wl0_agg']) + w['bl0']

    gate = (_dot(h.reshape(BI * L, 32), w['Wg']) + w['bg']).reshape(BI, L, 3)
    mg = mask3 * gate
    l1_full = xyz_all - ca_all[:, None, :]
    l1_blk = xyz_blk - ca_blk[:, None, :]

    avs = [jnp.sum(mg[:, :, a:a + 1] * l1_full[:, a, :][None, :, :], axis=1)
           for a in range(3)]
    agg_vec = jnp.stack(avs, axis=1) / float(TOP_K) + l1_blk

    vm = w['vecmix']
    Toff = (vm[0:1, 0:1] * agg_vec[:, 0, :] + vm[0:1, 1:2] * agg_vec[:, 1, :]
            + vm[0:1, 2:3] * agg_vec[:, 2, :]) / 10.0
    Roff = (vm[1:2, 0:1] * agg_vec[:, 0, :] + vm[1:2, 1:2] * agg_vec[:, 1, :]
            + vm[1:2, 2:3] * agg_vec[:, 2, :]) / 100.0
    Qn = jnp.sqrt(1.0 + jnp.sum(Roff * Roff, -1, keepdims=True))
    qA = 1.0 / Qn
    qB = Roff[:, 0:1] / Qn
    qC = Roff[:, 1:2] / Qn
    qD = Roff[:, 2:3] / Qn
    r00 = qA * qA + qB * qB - qC * qC - qD * qD
    r01 = 2 * qB * qC - 2 * qA * qD
    r02 = 2 * qB * qD + 2 * qA * qC
    r10 = 2 * qB * qC + 2 * qA * qD
    r11 = qA * qA - qB * qB + qC * qC - qD * qD
    r12 = 2 * qC * qD - 2 * qA * qB
    r20 = 2 * qB * qD - 2 * qA * qC
    r21 = 2 * qC * qD + 2 * qA * qB
    r22 = qA * qA - qB * qB - qC * qC + qD * qD
    v = l1_blk
    vx = v[:, :, 0]
    vy = v[:, :, 1]
    vz = v[:, :, 2]
    xn0 = r00 * vx + r01 * vy + r02 * vz
    xn1 = r10 * vx + r11 * vy + r12 * vz
    xn2 = r20 * vx + r21 * vy + r22 * vz
    xyz_new = (jnp.stack([xn0, xn1, xn2], axis=-1)
               + ca_blk[:, None, :] + Toff[:, None, :])
    oxyz_ref[:] = xyz_new
    ostate_ref[:] = state_out

    msa_blk = msa_ref[pl.ds(r0, BI), :]
    s = _ln(msa_blk, w['g_s0'], w['b_s0'])
    st2 = _ln(state_out, w['g_si'], w['b_si'])
    si = _dot(s, w['Ws0']) + w['bs0'] + _dot(st2, w['Wsi']) + w['bsi']
    si = si + _dot(jax.nn.relu(_dot(jax.nn.relu(si), w['W1']) + w['b1']),
                   w['W2']) + w['b2']
    si = si + _dot(jax.nn.relu(_dot(jax.nn.relu(si), w['W3']) + w['b3']),
                   w['W4']) + w['b4']
    oalpha_ref[:] = _dot(jax.nn.relu(si), w['Wo']) + w['bo']


@functools.partial(jax.jit, static_argnames=())
def _run(msa0, state0, xyz0, idxf, pair2, w):
    full = lambda a: pl.BlockSpec(a.shape, lambda i, nd=a.ndim: (0,) * nd)
    wspecs = jax.tree.map(full, w)
    grid = (L // BI,)
    out = pl.pallas_call(
        _body,
        grid=grid,
        in_specs=[full(msa0), full(state0), full(xyz0), full(idxf),
                  pl.BlockSpec((BI, L, 128), lambda i: (i, 0, 0)),
                  wspecs],
        out_specs=[pl.BlockSpec((BI, 3, 3), lambda i: (i, 0, 0)),
                   pl.BlockSpec((BI, 32), lambda i: (i, 0)),
                   pl.BlockSpec((BI, 20), lambda i: (i, 0))],
        out_shape=[jax.ShapeDtypeStruct((L, 3, 3), jnp.float32),
                   jax.ShapeDtypeStruct((L, 32), jnp.float32),
                   jax.ShapeDtypeStruct((L, 20), jnp.float32)],
        compiler_params=pltpu.CompilerParams(
            dimension_semantics=("arbitrary",),
            vmem_limit_bytes=120 * 2**20),
    )(msa0, state0, xyz0, idxf, pair2, w)
    return out


def kernel(msa, pair, xyz, state, idx, params):
    p = params
    msa0 = msa[0, 0]
    pair2 = pair[0]
    xyz0 = xyz[0]
    state0 = state[0]
    idxf = idx.astype(jnp.float32).reshape(L, 1)

    row = lambda a: a.reshape(1, -1)
    w = {
        'g_msa': row(p['norm_msa']['g']), 'b_msa': row(p['norm_msa']['b']),
        'g_pair': row(p['norm_pair']['g']), 'b_pair': row(p['norm_pair']['b']),
        'g_state': row(p['norm_state']['g']), 'b_state': row(p['norm_state']['b']),
        'Wx_m': p['embed_x']['W'][:, :256].T, 'Wx_s': p['embed_x']['W'][:, 256:].T,
        'bx': row(p['embed_x']['b']),
        'g_node': row(p['norm_node']['g']), 'b_node': row(p['norm_node']['b']),
        'W_e1': p['embed_e1']['W'].T, 'b_e1': row(p['embed_e1']['b']),
        'g_e1n': row(p['norm_edge1']['g']), 'b_e1n': row(p['norm_edge1']['b']),
        'W_e2_pr': p['embed_e2']['W'][:, :32].T,
        'W_e2_rbf': p['embed_e2']['W'][:, 32:68].T,
        'w_e2_nb': p['embed_e2']['W'][:, 68:].T,
        'b_e2': row(p['embed_e2']['b']),
        'g_e2n': row(p['norm_edge2']['g']), 'b_e2n': row(p['norm_edge2']['b']),
        'Wm_self': p['se3_msg']['W'][:, :32].T,
        'Wm_nbr': p['se3_msg']['W'][:, 32:64].T,
        'Wm_edge': p['se3_msg']['W'][:, 64:].T,
        'bm': row(p['se3_msg']['b']),
        'Wl0_node': p['se3_l0']['W'][:, :32].T,
        'Wl0_agg': p['se3_l0']['W'][:, 32:].T,
        'bl0': row(p['se3_l0']['b']),
        'Wg': p['se3_gate']['W'].T, 'bg': row(p['se3_gate']['b']),
        'vecmix': p['se3_vecmix'],
        'g_s0': row(p['sc_norm_s0']['g']), 'b_s0': row(p['sc_norm_s0']['b']),
        'g_si': row(p['sc_norm_si']['g']), 'b_si': row(p['sc_norm_si']['b']),
        'Ws0': p['sc_s0']['W'].T, 'bs0': row(p['sc_s0']['b']),
        'Wsi': p['sc_si']['W'].T, 'bsi': row(p['sc_si']['b']),
        'W1': p['sc_l1']['W'].T, 'b1': row(p['sc_l1']['b']),
        'W2': p['sc_l2']['W'].T, 'b2': row(p['sc_l2']['b']),
        'W3': p['sc_l3']['W'].T, 'b3': row(p['sc_l3']['b']),
        'W4': p['sc_l4']['W'].T, 'b4': row(p['sc_l4']['b']),
        'Wo': p['sc_out']['W'].T, 'bo': row(p['sc_out']['b']),
    }
    xyz_new, state_out, alpha = _run(msa0, state0, xyz0, idxf, pair2, w)
    return (xyz_new[None], state_out[None], alpha.reshape(1, L, 10, 2))


# speedup vs baseline: 4.6357x; 1.7581x over previous
import functools

import jax
import jax.numpy as jnp
from jax.experimental import pallas as pl
from jax.experimental.pallas import tpu as pltpu

L = 384
TOP_K = 128
BI = 32
EXP_BITS_INF = 0x7F800000


def _ln(x, g, b, eps=1e-5):
    g = g.reshape((1,) * (x.ndim - 1) + (-1,))
    b = b.reshape((1,) * (x.ndim - 1) + (-1,))
    m = jnp.mean(x, -1, keepdims=True)
    xc = x - m
    v = jnp.mean(xc * xc, -1, keepdims=True)
    return xc / jnp.sqrt(v + eps) * g + b


def _dot(x, w):
    return jnp.dot(x, w, preferred_element_type=jnp.float32)


def _body(msa_ref, state_ref, xyz_ref, idx_ref, pair_ref, w_ref,
          oxyz_ref, ostate_ref, oalpha_ref):
    i = pl.program_id(0)
    r0 = i * BI
    w = {k: v[:] for k, v in w_ref.items()}

    lnm = _ln(msa_ref[:], w['g_msa'], w['b_msa'])
    lnst = _ln(state_ref[:], w['g_state'], w['b_state'])
    node = _ln(_dot(lnm, w['Wx_m']) + _dot(lnst, w['Wx_s']) + w['bx'],
               w['g_node'], w['b_node'])

    xyz_all = xyz_ref[:]
    ca_all = xyz_all[:, 1, :]
    xyz_blk = xyz_ref[pl.ds(r0, BI), :, :]
    ca_blk = xyz_blk[:, 1, :]
    diff = ca_blk[:, None, :] - ca_all[None, :, :]
    D3 = jnp.sqrt(jnp.sum(diff * diff, -1, keepdims=True) + 1e-8)

    bits3 = jax.lax.bitcast_convert_type(D3, jnp.int32)
    bits_l = jnp.swapaxes(bits3, 1, 2).reshape(BI, L)
    lo0 = jnp.zeros((BI, 1), jnp.int32)
    hi0 = jnp.full((BI, 1), EXP_BITS_INF, jnp.int32)

    def bs_step(_, lohi):
        lo, hi = lohi
        mid = lo + ((hi - lo) >> 1)
        cnt = jnp.sum((bits_l <= mid).astype(jnp.float32), axis=1, keepdims=True)
        pred = cnt >= float(TOP_K)
        return jnp.where(pred, lo, mid), jnp.where(pred, mid, hi)

    _, T = jax.lax.fori_loop(0, 31, bs_step, (lo0, hi0))
    n_lt = jnp.sum((bits_l < T).astype(jnp.float32), axis=1, keepdims=True)
    coli_l = jax.lax.broadcasted_iota(jnp.int32, (1, L), 1)
    eqm_l = bits_l == T

    jlo0 = jnp.full((BI, 1), -1, jnp.int32)
    jhi0 = jnp.full((BI, 1), L - 1, jnp.int32)

    def js_step(_, lohi):
        lo, hi = lohi
        mid = lo + ((hi - lo) >> 1)
        cnt = n_lt + jnp.sum((eqm_l & (coli_l <= mid)).astype(jnp.float32),
                             axis=1, keepdims=True)
        pred = cnt >= float(TOP_K)
        return jnp.where(pred, lo, mid), jnp.where(pred, mid, hi)

    _, J = jax.lax.fori_loop(0, 9, js_step, (jlo0, jhi0))
    T3 = T.reshape(BI, 1, 1)
    J3 = J.reshape(BI, 1, 1)
    coli = jax.lax.broadcasted_iota(jnp.int32, (1, L, 1), 1)
    mask3 = ((bits3 < T3)
             | ((bits3 == T3) & (coli <= J3))).astype(jnp.float32)

    prn = _ln(pair_ref[:], w['g_pair'], w['b_pair'])
    pr1 = _ln(_dot(prn.reshape(BI * L, 128), w['W_e1']) + w['b_e1'],
              w['g_e1n'], w['b_e1n'])
    mu3 = 2.0 + jax.lax.broadcasted_iota(jnp.int32, (1, 1, 36), 2).astype(
        jnp.float32) * (20.0 / 35.0)
    rbf = jnp.exp(-jnp.square(D3 - mu3)).reshape(BI * L, 36)
    idx_all3 = idx_ref[:][None, :, :]
    idx_blk3 = idx_ref[pl.ds(r0, BI), :][:, :, None]
    nb = jnp.log(jnp.abs(idx_blk3 - idx_all3) + 1.0).reshape(BI * L, 1)
    e2 = (_dot(pr1, w['W_e2_pr']) + _dot(rbf, w['W_e2_rbf'])
          + nb * w['w_e2_nb'] + w['b_e2'])
    edge = _ln(e2, w['g_e2n'], w['b_e2n'])

    lnm_b = _ln(msa_ref[pl.ds(r0, BI), :], w['g_msa'], w['b_msa'])
    lnst_b = _ln(state_ref[pl.ds(r0, BI), :], w['g_state'], w['b_state'])
    node_blk = _ln(_dot(lnm_b, w['Wx_m']) + _dot(lnst_b, w['Wx_s']) + w['bx'],
                   w['g_node'], w['b_node'])
    t_self = _dot(node_blk, w['Wm_self'])
    t_nbr = _dot(node, w['Wm_nbr'])
    t_edge = _dot(edge, w['Wm_edge']).reshape(BI, L, 32)
    h = jax.nn.relu(t_edge + t_nbr[None, :, :] + t_self[:, None, :]
                    + w['bm'].reshape(1, 1, 32))

    aggm = jnp.sum(mask3 * h, axis=1) / float(TOP_K)
    state_out = _dot(node_blk, w['Wl0_node']) + _dot(aggm, w['Wl0_agg']) + w['bl0']

    gate = (_dot(h.reshape(BI * L, 32), w['Wg']) + w['bg']).reshape(BI, L, 3)
    mg = mask3 * gate
    l1_full = xyz_all - ca_all[:, None, :]
    l1_blk = xyz_blk - ca_blk[:, None, :]

    avs = [jnp.sum(mg[:, :, a:a + 1] * l1_full[:, a, :][None, :, :], axis=1)
           for a in range(3)]
    agg_vec = jnp.stack(avs, axis=1) / float(TOP_K) + l1_blk

    vm = w['vecmix']
    Toff = (vm[0:1, 0:1] * agg_vec[:, 0, :] + vm[0:1, 1:2] * agg_vec[:, 1, :]
            + vm[0:1, 2:3] * agg_vec[:, 2, :]) / 10.0
    Roff = (vm[1:2, 0:1] * agg_vec[:, 0, :] + vm[1:2, 1:2] * agg_vec[:, 1, :]
            + vm[1:2, 2:3] * agg_vec[:, 2, :]) / 100.0
    Qn = jnp.sqrt(1.0 + jnp.sum(Roff * Roff, -1, keepdims=True))
    qA = 1.0 / Qn
    qB = Roff[:, 0:1] / Qn
    qC = Roff[:, 1:2] / Qn
    qD = Roff[:, 2:3] / Qn
    r00 = qA * qA + qB * qB - qC * qC - qD * qD
    r01 = 2 * qB * qC - 2 * qA * qD
    r02 = 2 * qB * qD + 2 * qA * qC
    r10 = 2 * qB * qC + 2 * qA * qD
    r11 = qA * qA - qB * qB + qC * qC - qD * qD
    r12 = 2 * qC * qD - 2 * qA * qB
    r20 = 2 * qB * qD - 2 * qA * qC
    r21 = 2 * qC * qD + 2 * qA * qB
    r22 = qA * qA - qB * qB - qC * qC + qD * qD
    v = l1_blk
    vx = v[:, :, 0]
    vy = v[:, :, 1]
    vz = v[:, :, 2]
    xn0 = r00 * vx + r01 * vy + r02 * vz
    xn1 = r10 * vx + r11 * vy + r12 * vz
    xn2 = r20 * vx + r21 * vy + r22 * vz
    xyz_new = (jnp.stack([xn0, xn1, xn2], axis=-1)
               + ca_blk[:, None, :] + Toff[:, None, :])
    oxyz_ref[:] = xyz_new
    ostate_ref[:] = state_out

    msa_blk = msa_ref[pl.ds(r0, BI), :]
    s = _ln(msa_blk, w['g_s0'], w['b_s0'])
    st2 = _ln(state_out, w['g_si'], w['b_si'])
    si = _dot(s, w['Ws0']) + w['bs0'] + _dot(st2, w['Wsi']) + w['bsi']
    si = si + _dot(jax.nn.relu(_dot(jax.nn.relu(si), w['W1']) + w['b1']),
                   w['W2']) + w['b2']
    si = si + _dot(jax.nn.relu(_dot(jax.nn.relu(si), w['W3']) + w['b3']),
                   w['W4']) + w['b4']
    oalpha_ref[:] = _dot(jax.nn.relu(si), w['Wo']) + w['bo']


@functools.partial(jax.jit, static_argnames=())
def _run(msa0, state0, xyz0, idxf, pair2, w):
    full = lambda a: pl.BlockSpec(a.shape, lambda i, nd=a.ndim: (0,) * nd)
    wspecs = jax.tree.map(full, w)
    grid = (L // BI,)
    out = pl.pallas_call(
        _body,
        grid=grid,
        in_specs=[full(msa0), full(state0), full(xyz0), full(idxf),
                  pl.BlockSpec((BI, L, 128), lambda i: (i, 0, 0)),
                  wspecs],
        out_specs=[pl.BlockSpec((BI, 3, 3), lambda i: (i, 0, 0)),
                   pl.BlockSpec((BI, 32), lambda i: (i, 0)),
                   pl.BlockSpec((BI, 20), lambda i: (i, 0))],
        out_shape=[jax.ShapeDtypeStruct((L, 3, 3), jnp.float32),
                   jax.ShapeDtypeStruct((L, 32), jnp.float32),
                   jax.ShapeDtypeStruct((L, 20), jnp.float32)],
        compiler_params=pltpu.CompilerParams(
            dimension_semantics=("arbitrary",),
            vmem_limit_bytes=120 * 2**20),
    )(msa0, state0, xyz0, idxf, pair2, w)
    return out


def kernel(msa, pair, xyz, state, idx, params):
    p = params
    msa0 = msa[0, 0]
    pair2 = pair[0]
    xyz0 = xyz[0]
    state0 = state[0]
    idxf = idx.astype(jnp.float32).reshape(L, 1)

    row = lambda a: a.reshape(1, -1)
    w = {
        'g_msa': row(p['norm_msa']['g']), 'b_msa': row(p['norm_msa']['b']),
        'g_pair': row(p['norm_pair']['g']), 'b_pair': row(p['norm_pair']['b']),
        'g_state': row(p['norm_state']['g']), 'b_state': row(p['norm_state']['b']),
        'Wx_m': p['embed_x']['W'][:, :256].T, 'Wx_s': p['embed_x']['W'][:, 256:].T,
        'bx': row(p['embed_x']['b']),
        'g_node': row(p['norm_node']['g']), 'b_node': row(p['norm_node']['b']),
        'W_e1': p['embed_e1']['W'].T, 'b_e1': row(p['embed_e1']['b']),
        'g_e1n': row(p['norm_edge1']['g']), 'b_e1n': row(p['norm_edge1']['b']),
        'W_e2_pr': p['embed_e2']['W'][:, :32].T,
        'W_e2_rbf': p['embed_e2']['W'][:, 32:68].T,
        'w_e2_nb': p['embed_e2']['W'][:, 68:].T,
        'b_e2': row(p['embed_e2']['b']),
        'g_e2n': row(p['norm_edge2']['g']), 'b_e2n': row(p['norm_edge2']['b']),
        'Wm_self': p['se3_msg']['W'][:, :32].T,
        'Wm_nbr': p['se3_msg']['W'][:, 32:64].T,
        'Wm_edge': p['se3_msg']['W'][:, 64:].T,
        'bm': row(p['se3_msg']['b']),
        'Wl0_node': p['se3_l0']['W'][:, :32].T,
        'Wl0_agg': p['se3_l0']['W'][:, 32:].T,
        'bl0': row(p['se3_l0']['b']),
        'Wg': p['se3_gate']['W'].T, 'bg': row(p['se3_gate']['b']),
        'vecmix': p['se3_vecmix'],
        'g_s0': row(p['sc_norm_s0']['g']), 'b_s0': row(p['sc_norm_s0']['b']),
        'g_si': row(p['sc_norm_si']['g']), 'b_si': row(p['sc_norm_si']['b']),
        'Ws0': p['sc_s0']['W'].T, 'bs0': row(p['sc_s0']['b']),
        'Wsi': p['sc_si']['W'].T, 'bsi': row(p['sc_si']['b']),
        'W1': p['sc_l1']['W'].T, 'b1': row(p['sc_l1']['b']),
        'W2': p['sc_l2']['W'].T, 'b2': row(p['sc_l2']['b']),
        'W3': p['sc_l3']['W'].T, 'b3': row(p['sc_l3']['b']),
        'W4': p['sc_l4']['W'].T, 'b4': row(p['sc_l4']['b']),
        'Wo': p['sc_out']['W'].T, 'bo': row(p['sc_out']['b']),
    }
    xyz_new, state_out, alpha = _run(msa0, state0, xyz0, idxf, pair2, w)
    return (xyz_new[None], state_out[None], alpha.reshape(1, L, 10, 2))


# speedup vs baseline: 4.8266x; 1.0412x over previous
import functools

import jax
import jax.numpy as jnp
from jax.experimental import pallas as pl
from jax.experimental.pallas import tpu as pltpu

L = 384
TOP_K = 128
BI = 32
EXP_BITS_INF = 0x7F800000


def _ln(x, g, b, eps=1e-5):
    g = g.reshape((1,) * (x.ndim - 1) + (-1,))
    b = b.reshape((1,) * (x.ndim - 1) + (-1,))
    m = jnp.mean(x, -1, keepdims=True)
    xc = x - m
    v = jnp.mean(xc * xc, -1, keepdims=True)
    return xc / jnp.sqrt(v + eps) * g + b


def _dot(x, w):
    return jnp.dot(x, w, preferred_element_type=jnp.float32)


def _body(msa_ref, state_ref, xyz_ref, idx_ref, pair_ref, w_ref,
          oxyz_ref, ostate_ref, oalpha_ref):
    i = pl.program_id(0)
    r0 = i * BI
    w = {k: v[:] for k, v in w_ref.items()}

    lnm = _ln(msa_ref[:], w['g_msa'], w['b_msa'])
    lnst = _ln(state_ref[:], w['g_state'], w['b_state'])
    node = _ln(_dot(lnm, w['Wx_m']) + _dot(lnst, w['Wx_s']) + w['bx'],
               w['g_node'], w['b_node'])

    xyz_all = xyz_ref[:]
    ca_all = xyz_all[:, 1, :]
    xyz_blk = xyz_ref[pl.ds(r0, BI), :, :]
    ca_blk = xyz_blk[:, 1, :]
    diff = ca_blk[:, None, :] - ca_all[None, :, :]
    D3 = jnp.sqrt(jnp.sum(diff * diff, -1, keepdims=True) + 1e-8)

    bits3 = jax.lax.bitcast_convert_type(D3, jnp.int32)
    bits_l = jnp.swapaxes(bits3, 1, 2).reshape(BI, L)
    lo0 = jnp.zeros((BI, 1), jnp.int32)
    hi0 = jnp.full((BI, 1), EXP_BITS_INF, jnp.int32)

    def bs_step(_, lohi):
        lo, hi = lohi
        mid = lo + ((hi - lo) >> 1)
        cnt = jnp.sum((bits_l <= mid).astype(jnp.float32), axis=1, keepdims=True)
        pred = cnt >= float(TOP_K)
        return jnp.where(pred, lo, mid), jnp.where(pred, mid, hi)

    _, T = jax.lax.fori_loop(0, 31, bs_step, (lo0, hi0))
    n_lt = jnp.sum((bits_l < T).astype(jnp.float32), axis=1, keepdims=True)
    coli_l = jax.lax.broadcasted_iota(jnp.int32, (1, L), 1)
    eqm_l = bits_l == T

    jlo0 = jnp.full((BI, 1), -1, jnp.int32)
    jhi0 = jnp.full((BI, 1), L - 1, jnp.int32)

    def js_step(_, lohi):
        lo, hi = lohi
        mid = lo + ((hi - lo) >> 1)
        cnt = n_lt + jnp.sum((eqm_l & (coli_l <= mid)).astype(jnp.float32),
                             axis=1, keepdims=True)
        pred = cnt >= float(TOP_K)
        return jnp.where(pred, lo, mid), jnp.where(pred, mid, hi)

    _, J = jax.lax.fori_loop(0, 9, js_step, (jlo0, jhi0))
    T3 = T.reshape(BI, 1, 1)
    J3 = J.reshape(BI, 1, 1)
    coli = jax.lax.broadcasted_iota(jnp.int32, (1, L, 1), 1)
    mask3 = ((bits3 < T3)
             | ((bits3 == T3) & (coli <= J3))).astype(jnp.float32)

    P = pair_ref[:].reshape(BI * L, 128)
    m = jnp.sum(P, -1, keepdims=True) * (1.0 / 128.0)
    msq = jnp.sum(P * P, -1, keepdims=True) * (1.0 / 128.0)
    rs = jax.lax.rsqrt(msq - m * m + 1e-5)
    S1 = _dot(P, w['W1g'])
    pr1 = (S1 - m * w['w1sum']) * rs + w['be1tot']
    m1 = jnp.mean(pr1, -1, keepdims=True)
    rs1 = jax.lax.rsqrt(jnp.mean(pr1 * pr1, -1, keepdims=True)
                        - m1 * m1 + 1e-5)
    pr1c = (pr1 - m1) * rs1
    mu3 = 2.0 + jax.lax.broadcasted_iota(jnp.int32, (1, 1, 36), 2).astype(
        jnp.float32) * (20.0 / 35.0)
    rbf = jnp.exp(-jnp.square(D3 - mu3)).reshape(BI * L, 36)
    idx_all3 = idx_ref[:][None, :, :]
    idx_blk3 = idx_ref[pl.ds(r0, BI), :][:, :, None]
    nb = jnp.log(jnp.abs(idx_blk3 - idx_all3) + 1.0).reshape(BI * L, 1)
    X = jnp.concatenate([pr1c, rbf, nb], axis=-1)
    e2 = _dot(X, w['W2cat']) + w['be2tot']
    m2 = jnp.mean(e2, -1, keepdims=True)
    rs2 = jax.lax.rsqrt(jnp.mean(e2 * e2, -1, keepdims=True)
                        - m2 * m2 + 1e-5)
    edge = (e2 - m2) * rs2

    lnm_b = _ln(msa_ref[pl.ds(r0, BI), :], w['g_msa'], w['b_msa'])
    lnst_b = _ln(state_ref[pl.ds(r0, BI), :], w['g_state'], w['b_state'])
    node_blk = _ln(_dot(lnm_b, w['Wx_m']) + _dot(lnst_b, w['Wx_s']) + w['bx'],
                   w['g_node'], w['b_node'])
    t_self = _dot(node_blk, w['Wm_self'])
    t_nbr = _dot(node, w['Wm_nbr'])
    t_edge = _dot(edge, w['Wm_edge']).reshape(BI, L, 32)
    h = jax.nn.relu(t_edge + t_nbr[None, :, :] + t_self[:, None, :]
                    + w['bm'].reshape(1, 1, 32))

    aggm = jnp.sum(mask3 * h, axis=1) / float(TOP_K)
    state_out = _dot(node_blk, w['Wl0_node']) + _dot(aggm, w['Wl0_agg']) + w['bl0']

    gate = (_dot(h.reshape(BI * L, 32), w['Wg']) + w['bg']).reshape(BI, L, 3)
    mg = mask3 * gate
    l1_full = xyz_all - ca_all[:, None, :]
    l1_blk = xyz_blk - ca_blk[:, None, :]

    avs = [jnp.sum(mg[:, :, a:a + 1] * l1_full[:, a, :][None, :, :], axis=1)
           for a in range(3)]
    agg_vec = jnp.stack(avs, axis=1) / float(TOP_K) + l1_blk

    vm = w['vecmix']
    Toff = (vm[0:1, 0:1] * agg_vec[:, 0, :] + vm[0:1, 1:2] * agg_vec[:, 1, :]
            + vm[0:1, 2:3] * agg_vec[:, 2, :]) / 10.0
    Roff = (vm[1:2, 0:1] * agg_vec[:, 0, :] + vm[1:2, 1:2] * agg_vec[:, 1, :]
            + vm[1:2, 2:3] * agg_vec[:, 2, :]) / 100.0
    Qn = jnp.sqrt(1.0 + jnp.sum(Roff * Roff, -1, keepdims=True))
    qA = 1.0 / Qn
    qB = Roff[:, 0:1] / Qn
    qC = Roff[:, 1:2] / Qn
    qD = Roff[:, 2:3] / Qn
    r00 = qA * qA + qB * qB - qC * qC - qD * qD
    r01 = 2 * qB * qC - 2 * qA * qD
    r02 = 2 * qB * qD + 2 * qA * qC
    r10 = 2 * qB * qC + 2 * qA * qD
    r11 = qA * qA - qB * qB + qC * qC - qD * qD
    r12 = 2 * qC * qD - 2 * qA * qB
    r20 = 2 * qB * qD - 2 * qA * qC
    r21 = 2 * qC * qD + 2 * qA * qB
    r22 = qA * qA - qB * qB - qC * qC + qD * qD
    v = l1_blk
    vx = v[:, :, 0]
    vy = v[:, :, 1]
    vz = v[:, :, 2]
    xn0 = r00 * vx + r01 * vy + r02 * vz
    xn1 = r10 * vx + r11 * vy + r12 * vz
    xn2 = r20 * vx + r21 * vy + r22 * vz
    xyz_new = (jnp.stack([xn0, xn1, xn2], axis=-1)
               + ca_blk[:, None, :] + Toff[:, None, :])
    oxyz_ref[:] = xyz_new
    ostate_ref[:] = state_out

    msa_blk = msa_ref[pl.ds(r0, BI), :]
    s = _ln(msa_blk, w['g_s0'], w['b_s0'])
    st2 = _ln(state_out, w['g_si'], w['b_si'])
    si = _dot(s, w['Ws0']) + w['bs0'] + _dot(st2, w['Wsi']) + w['bsi']
    si = si + _dot(jax.nn.relu(_dot(jax.nn.relu(si), w['W1']) + w['b1']),
                   w['W2']) + w['b2']
    si = si + _dot(jax.nn.relu(_dot(jax.nn.relu(si), w['W3']) + w['b3']),
                   w['W4']) + w['b4']
    oalpha_ref[:] = _dot(jax.nn.relu(si), w['Wo']) + w['bo']


@functools.partial(jax.jit, static_argnames=())
def _run(msa0, state0, xyz0, idxf, pair2, w):
    full = lambda a: pl.BlockSpec(a.shape, lambda i, nd=a.ndim: (0,) * nd)
    wspecs = jax.tree.map(full, w)
    grid = (L // BI,)
    out = pl.pallas_call(
        _body,
        grid=grid,
        in_specs=[full(msa0), full(state0), full(xyz0), full(idxf),
                  pl.BlockSpec((BI, L, 128), lambda i: (i, 0, 0)),
                  wspecs],
        out_specs=[pl.BlockSpec((BI, 3, 3), lambda i: (i, 0, 0)),
                   pl.BlockSpec((BI, 32), lambda i: (i, 0)),
                   pl.BlockSpec((BI, 20), lambda i: (i, 0))],
        out_shape=[jax.ShapeDtypeStruct((L, 3, 3), jnp.float32),
                   jax.ShapeDtypeStruct((L, 32), jnp.float32),
                   jax.ShapeDtypeStruct((L, 20), jnp.float32)],
        compiler_params=pltpu.CompilerParams(
            dimension_semantics=("arbitrary",),
            vmem_limit_bytes=120 * 2**20),
    )(msa0, state0, xyz0, idxf, pair2, w)
    return out


def kernel(msa, pair, xyz, state, idx, params):
    p = params
    msa0 = msa[0, 0]
    pair2 = pair[0]
    xyz0 = xyz[0]
    state0 = state[0]
    idxf = idx.astype(jnp.float32).reshape(L, 1)

    row = lambda a: a.reshape(1, -1)
    W1 = p['embed_e1']['W']
    g_p, b_p = p['norm_pair']['g'], p['norm_pair']['b']
    W1g = W1.T * g_p[:, None]
    g1, b1n = p['norm_edge1']['g'], p['norm_edge1']['b']
    W2T = p['embed_e2']['W'].T
    W2cat = jnp.concatenate([W2T[:32] * g1[:, None], W2T[32:]], axis=0)
    g2, b2n = p['norm_edge2']['g'], p['norm_edge2']['b']
    WmE_T = p['se3_msg']['W'][:, 64:].T
    w = {
        'g_msa': row(p['norm_msa']['g']), 'b_msa': row(p['norm_msa']['b']),
        'g_state': row(p['norm_state']['g']), 'b_state': row(p['norm_state']['b']),
        'Wx_m': p['embed_x']['W'][:, :256].T, 'Wx_s': p['embed_x']['W'][:, 256:].T,
        'bx': row(p['embed_x']['b']),
        'g_node': row(p['norm_node']['g']), 'b_node': row(p['norm_node']['b']),
        'W1g': W1g,
        'w1sum': jnp.sum(W1g, axis=0, keepdims=True),
        'be1tot': row(b_p @ W1.T + p['embed_e1']['b']),
        'W2cat': W2cat,
        'be2tot': row(b1n @ W2T[:32] + p['embed_e2']['b']),
        'Wm_self': p['se3_msg']['W'][:, :32].T,
        'Wm_nbr': p['se3_msg']['W'][:, 32:64].T,
        'Wm_edge': WmE_T * g2[:, None],
        'bm': row(p['se3_msg']['b'] + b2n @ WmE_T),
        'Wl0_node': p['se3_l0']['W'][:, :32].T,
        'Wl0_agg': p['se3_l0']['W'][:, 32:].T,
        'bl0': row(p['se3_l0']['b']),
        'Wg': p['se3_gate']['W'].T, 'bg': row(p['se3_gate']['b']),
        'vecmix': p['se3_vecmix'],
        'g_s0': row(p['sc_norm_s0']['g']), 'b_s0': row(p['sc_norm_s0']['b']),
        'g_si': row(p['sc_norm_si']['g']), 'b_si': row(p['sc_norm_si']['b']),
        'Ws0': p['sc_s0']['W'].T, 'bs0': row(p['sc_s0']['b']),
        'Wsi': p['sc_si']['W'].T, 'bsi': row(p['sc_si']['b']),
        'W1': p['sc_l1']['W'].T, 'b1': row(p['sc_l1']['b']),
        'W2': p['sc_l2']['W'].T, 'b2': row(p['sc_l2']['b']),
        'W3': p['sc_l3']['W'].T, 'b3': row(p['sc_l3']['b']),
        'W4': p['sc_l4']['W'].T, 'b4': row(p['sc_l4']['b']),
        'Wo': p['sc_out']['W'].T, 'bo': row(p['sc_out']['b']),
    }
    xyz_new, state_out, alpha = _run(msa0, state0, xyz0, idxf, pair2, w)
    return (xyz_new[None], state_out[None], alpha.reshape(1, L, 10, 2))


# speedup vs baseline: 4.9210x; 1.0195x over previous
import functools

import jax
import jax.numpy as jnp
from jax.experimental import pallas as pl
from jax.experimental.pallas import tpu as pltpu

L = 384
TOP_K = 128
BI = 32
EXP_BITS_INF = 0x7F800000


def _ln(x, g, b, eps=1e-5):
    g = g.reshape((1,) * (x.ndim - 1) + (-1,))
    b = b.reshape((1,) * (x.ndim - 1) + (-1,))
    m = jnp.mean(x, -1, keepdims=True)
    xc = x - m
    v = jnp.mean(xc * xc, -1, keepdims=True)
    return xc / jnp.sqrt(v + eps) * g + b


def _dot(x, w):
    return jnp.dot(x, w, preferred_element_type=jnp.float32)


def _body(msa_ref, state_ref, xyz_ref, caT_ref, idxc_ref, idxr_ref, pair_ref,
          w_ref, oxyz_ref, ostate_ref, oalpha_ref):
    i = pl.program_id(0)
    r0 = i * BI
    w = {k: v[:] for k, v in w_ref.items()}

    lnm = _ln(msa_ref[:], w['g_msa'], w['b_msa'])
    lnst = _ln(state_ref[:], w['g_state'], w['b_state'])
    node = _ln(_dot(lnm, w['Wx_m']) + _dot(lnst, w['Wx_s']) + w['bx'],
               w['g_node'], w['b_node'])

    xyz_all = xyz_ref[:]
    ca_all = xyz_all[:, 1, :]
    xyz_blk = xyz_ref[pl.ds(r0, BI), :, :]
    ca_blk = xyz_blk[:, 1, :]
    dx = ca_blk[:, 0:1] - caT_ref[0:1, :]
    dy = ca_blk[:, 1:2] - caT_ref[1:2, :]
    dz = ca_blk[:, 2:3] - caT_ref[2:3, :]
    D_l = jnp.sqrt(((dx * dx + dy * dy) + dz * dz) + 1e-8)

    bits_l = jax.lax.bitcast_convert_type(D_l, jnp.int32)
    lo0 = jnp.zeros((BI, 1), jnp.int32)
    hi0 = jnp.full((BI, 1), EXP_BITS_INF, jnp.int32)

    def bs_step(_, lohi):
        lo, hi = lohi
        mid = lo + ((hi - lo) >> 1)
        cnt = jnp.sum((bits_l <= mid).astype(jnp.float32), axis=1, keepdims=True)
        pred = cnt >= float(TOP_K)
        return jnp.where(pred, lo, mid), jnp.where(pred, mid, hi)

    _, T = jax.lax.fori_loop(0, 31, bs_step, (lo0, hi0))
    n_lt = jnp.sum((bits_l < T).astype(jnp.float32), axis=1, keepdims=True)
    coli_l = jax.lax.broadcasted_iota(jnp.int32, (1, L), 1)
    eqm_l = bits_l == T

    jlo0 = jnp.full((BI, 1), -1, jnp.int32)
    jhi0 = jnp.full((BI, 1), L - 1, jnp.int32)

    def js_step(_, lohi):
        lo, hi = lohi
        mid = lo + ((hi - lo) >> 1)
        cnt = n_lt + jnp.sum((eqm_l & (coli_l <= mid)).astype(jnp.float32),
                             axis=1, keepdims=True)
        pred = cnt >= float(TOP_K)
        return jnp.where(pred, lo, mid), jnp.where(pred, mid, hi)

    _, J = jax.lax.fori_loop(0, 9, js_step, (jlo0, jhi0))
    mask_l = ((bits_l < T)
              | (eqm_l & (coli_l <= J))).astype(jnp.float32)

    nb_l = jnp.log(jnp.abs(idxc_ref[pl.ds(r0, BI), :] - idxr_ref[0:1, :]) + 1.0)

    packT = jnp.swapaxes(jnp.stack([D_l, nb_l, mask_l], axis=1), 1, 2)
    D3 = packT[:, :, 0:1]
    nb3 = packT[:, :, 1:2]
    mask3 = packT[:, :, 2:3]

    P = pair_ref[:].reshape(BI * L, 128)
    m = jnp.sum(P, -1, keepdims=True) * (1.0 / 128.0)
    msq = jnp.sum(P * P, -1, keepdims=True) * (1.0 / 128.0)
    rs = jax.lax.rsqrt(msq - m * m + 1e-5)
    S1 = _dot(P, w['W1g'])
    pr1 = (S1 - m * w['w1sum']) * rs + w['be1tot']
    m1 = jnp.mean(pr1, -1, keepdims=True)
    rs1 = jax.lax.rsqrt(jnp.mean(pr1 * pr1, -1, keepdims=True)
                        - m1 * m1 + 1e-5)
    pr1c = (pr1 - m1) * rs1
    mu3 = 2.0 + jax.lax.broadcasted_iota(jnp.int32, (1, 1, 36), 2).astype(
        jnp.float32) * (20.0 / 35.0)
    rbf = jnp.exp(-jnp.square(D3 - mu3)).reshape(BI * L, 36)
    nb = nb3.reshape(BI * L, 1)
    X = jnp.concatenate([pr1c, rbf, nb], axis=-1)
    e2 = _dot(X, w['W2cat']) + w['be2tot']
    m2 = jnp.mean(e2, -1, keepdims=True)
    rs2 = jax.lax.rsqrt(jnp.mean(e2 * e2, -1, keepdims=True)
                        - m2 * m2 + 1e-5)
    edge = (e2 - m2) * rs2

    lnm_b = _ln(msa_ref[pl.ds(r0, BI), :], w['g_msa'], w['b_msa'])
    lnst_b = _ln(state_ref[pl.ds(r0, BI), :], w['g_state'], w['b_state'])
    node_blk = _ln(_dot(lnm_b, w['Wx_m']) + _dot(lnst_b, w['Wx_s']) + w['bx'],
                   w['g_node'], w['b_node'])
    t_self = _dot(node_blk, w['Wm_self'])
    t_nbr = _dot(node, w['Wm_nbr'])
    t_edge = _dot(edge, w['Wm_edge']).reshape(BI, L, 32)
    h = jax.nn.relu(t_edge + t_nbr[None, :, :] + t_self[:, None, :]
                    + w['bm'].reshape(1, 1, 32)
                    + (mask3 - 1.0) * 1e30)
    aggm = jnp.sum(h, axis=1) / float(TOP_K)
    state_out = _dot(node_blk, w['Wl0_node']) + _dot(aggm, w['Wl0_agg']) + w['bl0']

    gate0 = _dot(h.reshape(BI * L, 32), w['Wg']).reshape(BI, L, 3)
    l1_full = xyz_all - ca_all[:, None, :]
    l1_blk = xyz_blk - ca_blk[:, None, :]

    avs = [jnp.sum(gate0[:, :, a:a + 1] * l1_full[:, a, :][None, :, :], axis=1)
           + w['bg'][0:1, a:a + 1] * _dot(mask_l, l1_full[:, a, :])
           for a in range(3)]
    agg_vec = jnp.stack(avs, axis=1) / float(TOP_K) + l1_blk

    vm = w['vecmix']
    Toff = (vm[0:1, 0:1] * agg_vec[:, 0, :] + vm[0:1, 1:2] * agg_vec[:, 1, :]
            + vm[0:1, 2:3] * agg_vec[:, 2, :]) / 10.0
    Roff = (vm[1:2, 0:1] * agg_vec[:, 0, :] + vm[1:2, 1:2] * agg_vec[:, 1, :]
            + vm[1:2, 2:3] * agg_vec[:, 2, :]) / 100.0
    Qn = jnp.sqrt(1.0 + jnp.sum(Roff * Roff, -1, keepdims=True))
    qA = 1.0 / Qn
    qB = Roff[:, 0:1] / Qn
    qC = Roff[:, 1:2] / Qn
    qD = Roff[:, 2:3] / Qn
    r00 = qA * qA + qB * qB - qC * qC - qD * qD
    r01 = 2 * qB * qC - 2 * qA * qD
    r02 = 2 * qB * qD + 2 * qA * qC
    r10 = 2 * qB * qC + 2 * qA * qD
    r11 = qA * qA - qB * qB + qC * qC - qD * qD
    r12 = 2 * qC * qD - 2 * qA * qB
    r20 = 2 * qB * qD - 2 * qA * qC
    r21 = 2 * qC * qD + 2 * qA * qB
    r22 = qA * qA - qB * qB - qC * qC + qD * qD
    v = l1_blk
    vx = v[:, :, 0]
    vy = v[:, :, 1]
    vz = v[:, :, 2]
    xn0 = r00 * vx + r01 * vy + r02 * vz
    xn1 = r10 * vx + r11 * vy + r12 * vz
    xn2 = r20 * vx + r21 * vy + r22 * vz
    xyz_new = (jnp.stack([xn0, xn1, xn2], axis=-1)
               + ca_blk[:, None, :] + Toff[:, None, :])
    oxyz_ref[:] = xyz_new
    ostate_ref[:] = state_out

    msa_blk = msa_ref[pl.ds(r0, BI), :]
    s = _ln(msa_blk, w['g_s0'], w['b_s0'])
    st2 = _ln(state_out, w['g_si'], w['b_si'])
    si = _dot(s, w['Ws0']) + w['bs0'] + _dot(st2, w['Wsi']) + w['bsi']
    si = si + _dot(jax.nn.relu(_dot(jax.nn.relu(si), w['W1']) + w['b1']),
                   w['W2']) + w['b2']
    si = si + _dot(jax.nn.relu(_dot(jax.nn.relu(si), w['W3']) + w['b3']),
                   w['W4']) + w['b4']
    oalpha_ref[:] = _dot(jax.nn.relu(si), w['Wo']) + w['bo']


@functools.partial(jax.jit, static_argnames=())
def _run(msa0, state0, xyz0, caT, idxc, idxr, pair2, w):
    full = lambda a: pl.BlockSpec(a.shape, lambda i, nd=a.ndim: (0,) * nd)
    wspecs = jax.tree.map(full, w)
    grid = (L // BI,)
    out = pl.pallas_call(
        _body,
        grid=grid,
        in_specs=[full(msa0), full(state0), full(xyz0), full(caT),
                  full(idxc), full(idxr),
                  pl.BlockSpec((BI, L, 128), lambda i: (i, 0, 0)),
                  wspecs],
        out_specs=[pl.BlockSpec((BI, 3, 3), lambda i: (i, 0, 0)),
                   pl.BlockSpec((BI, 32), lambda i: (i, 0)),
                   pl.BlockSpec((BI, 20), lambda i: (i, 0))],
        out_shape=[jax.ShapeDtypeStruct((L, 3, 3), jnp.float32),
                   jax.ShapeDtypeStruct((L, 32), jnp.float32),
                   jax.ShapeDtypeStruct((L, 20), jnp.float32)],
        compiler_params=pltpu.CompilerParams(
            dimension_semantics=("arbitrary",),
            vmem_limit_bytes=120 * 2**20),
    )(msa0, state0, xyz0, caT, idxc, idxr, pair2, w)
    return out


def kernel(msa, pair, xyz, state, idx, params):
    p = params
    msa0 = msa[0, 0]
    pair2 = pair[0]
    xyz0 = xyz[0]
    state0 = state[0]
    caT = xyz0[:, 1, :].T
    idxc = idx.astype(jnp.float32).reshape(L, 1)
    idxr = idx.astype(jnp.float32).reshape(1, L)

    row = lambda a: a.reshape(1, -1)
    W1 = p['embed_e1']['W']
    g_p, b_p = p['norm_pair']['g'], p['norm_pair']['b']
    W1g = W1.T * g_p[:, None]
    g1, b1n = p['norm_edge1']['g'], p['norm_edge1']['b']
    W2T = p['embed_e2']['W'].T
    W2cat = jnp.concatenate([W2T[:32] * g1[:, None], W2T[32:]], axis=0)
    g2, b2n = p['norm_edge2']['g'], p['norm_edge2']['b']
    WmE_T = p['se3_msg']['W'][:, 64:].T
    w = {
        'g_msa': row(p['norm_msa']['g']), 'b_msa': row(p['norm_msa']['b']),
        'g_state': row(p['norm_state']['g']), 'b_state': row(p['norm_state']['b']),
        'Wx_m': p['embed_x']['W'][:, :256].T, 'Wx_s': p['embed_x']['W'][:, 256:].T,
        'bx': row(p['embed_x']['b']),
        'g_node': row(p['norm_node']['g']), 'b_node': row(p['norm_node']['b']),
        'W1g': W1g,
        'w1sum': jnp.sum(W1g, axis=0, keepdims=True),
        'be1tot': row(b_p @ W1.T + p['embed_e1']['b']),
        'W2cat': W2cat,
        'be2tot': row(b1n @ W2T[:32] + p['embed_e2']['b']),
        'Wm_self': p['se3_msg']['W'][:, :32].T,
        'Wm_nbr': p['se3_msg']['W'][:, 32:64].T,
        'Wm_edge': WmE_T * g2[:, None],
        'bm': row(p['se3_msg']['b'] + b2n @ WmE_T),
        'Wl0_node': p['se3_l0']['W'][:, :32].T,
        'Wl0_agg': p['se3_l0']['W'][:, 32:].T,
        'bl0': row(p['se3_l0']['b']),
        'Wg': p['se3_gate']['W'].T, 'bg': row(p['se3_gate']['b']),
        'vecmix': p['se3_vecmix'],
        'g_s0': row(p['sc_norm_s0']['g']), 'b_s0': row(p['sc_norm_s0']['b']),
        'g_si': row(p['sc_norm_si']['g']), 'b_si': row(p['sc_norm_si']['b']),
        'Ws0': p['sc_s0']['W'].T, 'bs0': row(p['sc_s0']['b']),
        'Wsi': p['sc_si']['W'].T, 'bsi': row(p['sc_si']['b']),
        'W1': p['sc_l1']['W'].T, 'b1': row(p['sc_l1']['b']),
        'W2': p['sc_l2']['W'].T, 'b2': row(p['sc_l2']['b']),
        'W3': p['sc_l3']['W'].T, 'b3': row(p['sc_l3']['b']),
        'W4': p['sc_l4']['W'].T, 'b4': row(p['sc_l4']['b']),
        'Wo': p['sc_out']['W'].T, 'bo': row(p['sc_out']['b']),
    }
    xyz_new, state_out, alpha = _run(msa0, state0, xyz0, caT, idxc, idxr,
                                     pair2, w)
    return (xyz_new[None], state_out[None], alpha.reshape(1, L, 10, 2))


# speedup vs baseline: 5.4692x; 1.1114x over previous
import functools

import jax
import jax.numpy as jnp
from jax.experimental import pallas as pl
from jax.experimental.pallas import tpu as pltpu

L = 384
TOP_K = 128
BI = 32
EXP_BITS_INF = 0x7F800000


def _ln(x, g, b, eps=1e-5):
    g = g.reshape((1,) * (x.ndim - 1) + (-1,))
    b = b.reshape((1,) * (x.ndim - 1) + (-1,))
    m = jnp.mean(x, -1, keepdims=True)
    xc = x - m
    v = jnp.mean(xc * xc, -1, keepdims=True)
    return xc / jnp.sqrt(v + eps) * g + b


def _dot(x, w):
    return jnp.dot(x, w, preferred_element_type=jnp.float32)


def _body(msa_ref, state_ref, xyz_ref, caT_ref, idxc_ref, idxr_ref, pair_ref,
          w_ref, oxyz_ref, ostate_ref, oalpha_ref):
    i = pl.program_id(0)
    r0 = i * BI
    w = {k: v[:] for k, v in w_ref.items()}

    lnm = _ln(msa_ref[:], w['g_msa'], w['b_msa'])
    lnst = _ln(state_ref[:], w['g_state'], w['b_state'])
    node = _ln(_dot(lnm, w['Wx_m']) + _dot(lnst, w['Wx_s']) + w['bx'],
               w['g_node'], w['b_node'])

    xyz_all = xyz_ref[:]
    ca_all = xyz_all[:, 1, :]
    xyz_blk = xyz_ref[pl.ds(r0, BI), :, :]
    ca_blk = xyz_blk[:, 1, :]
    dx = ca_blk[:, 0:1] - caT_ref[0:1, :]
    dy = ca_blk[:, 1:2] - caT_ref[1:2, :]
    dz = ca_blk[:, 2:3] - caT_ref[2:3, :]
    D_l = jnp.sqrt(((dx * dx + dy * dy) + dz * dz) + 1e-8)

    bits_l = jax.lax.bitcast_convert_type(D_l, jnp.int32)
    lo0 = jnp.zeros((BI, 1), jnp.int32)
    hi0 = jnp.full((BI, 1), EXP_BITS_INF, jnp.int32)

    def bs_step(lo, hi):
        mid = lo + ((hi - lo) >> 1)
        cnt = jnp.sum((bits_l <= mid).astype(jnp.float32), axis=1, keepdims=True)
        pred = cnt >= float(TOP_K)
        return jnp.where(pred, lo, mid), jnp.where(pred, mid, hi)

    lo, hi = lo0, hi0
    for _ in range(31):
        lo, hi = bs_step(lo, hi)
    T = hi
    n_lt = jnp.sum((bits_l < T).astype(jnp.float32), axis=1, keepdims=True)
    coli_l = jax.lax.broadcasted_iota(jnp.int32, (1, L), 1)
    eqm_l = bits_l == T

    jlo0 = jnp.full((BI, 1), -1, jnp.int32)
    jhi0 = jnp.full((BI, 1), L - 1, jnp.int32)

    def js_step(lo, hi):
        mid = lo + ((hi - lo) >> 1)
        cnt = n_lt + jnp.sum((eqm_l & (coli_l <= mid)).astype(jnp.float32),
                             axis=1, keepdims=True)
        pred = cnt >= float(TOP_K)
        return jnp.where(pred, lo, mid), jnp.where(pred, mid, hi)

    jlo, jhi = jlo0, jhi0
    for _ in range(9):
        jlo, jhi = js_step(jlo, jhi)
    J = jhi
    mask_l = ((bits_l < T)
              | (eqm_l & (coli_l <= J))).astype(jnp.float32)

    nb_l = jnp.log(jnp.abs(idxc_ref[pl.ds(r0, BI), :] - idxr_ref[0:1, :]) + 1.0)

    packT = jnp.swapaxes(jnp.stack([D_l, nb_l, mask_l], axis=1), 1, 2)
    D3 = packT[:, :, 0:1]
    nb3 = packT[:, :, 1:2]
    mask3 = packT[:, :, 2:3]

    P = pair_ref[:].reshape(BI * L, 128)
    m = jnp.sum(P, -1, keepdims=True) * (1.0 / 128.0)
    msq = jnp.sum(P * P, -1, keepdims=True) * (1.0 / 128.0)
    rs = jax.lax.rsqrt(msq - m * m + 1e-5)
    S1 = _dot(P, w['W1g'])
    pr1 = (S1 - m * w['w1sum']) * rs + w['be1tot']
    m1 = jnp.mean(pr1, -1, keepdims=True)
    rs1 = jax.lax.rsqrt(jnp.mean(pr1 * pr1, -1, keepdims=True)
                        - m1 * m1 + 1e-5)
    pr1c = (pr1 - m1) * rs1
    mu3 = 2.0 + jax.lax.broadcasted_iota(jnp.int32, (1, 1, 36), 2).astype(
        jnp.float32) * (20.0 / 35.0)
    rbf = jnp.exp(-jnp.square(D3 - mu3)).reshape(BI * L, 36)
    nb = nb3.reshape(BI * L, 1)
    X = jnp.concatenate([pr1c, rbf, nb], axis=-1)
    e2 = _dot(X, w['W2cat']) + w['be2tot']
    m2 = jnp.mean(e2, -1, keepdims=True)
    rs2 = jax.lax.rsqrt(jnp.mean(e2 * e2, -1, keepdims=True)
                        - m2 * m2 + 1e-5)
    edge = (e2 - m2) * rs2

    lnm_b = _ln(msa_ref[pl.ds(r0, BI), :], w['g_msa'], w['b_msa'])
    lnst_b = _ln(state_ref[pl.ds(r0, BI), :], w['g_state'], w['b_state'])
    node_blk = _ln(_dot(lnm_b, w['Wx_m']) + _dot(lnst_b, w['Wx_s']) + w['bx'],
                   w['g_node'], w['b_node'])
    t_self = _dot(node_blk, w['Wm_self'])
    t_nbr = _dot(node, w['Wm_nbr'])
    t_edge = _dot(edge, w['Wm_edge']).reshape(BI, L, 32)
    h = jax.nn.relu(t_edge + t_nbr[None, :, :] + t_self[:, None, :]
                    + w['bm'].reshape(1, 1, 32)
                    + (mask3 - 1.0) * 1e30)
    aggm = jnp.sum(h, axis=1) / float(TOP_K)
    state_out = _dot(node_blk, w['Wl0_node']) + _dot(aggm, w['Wl0_agg']) + w['bl0']

    gate0 = _dot(h.reshape(BI * L, 32), w['Wg']).reshape(BI, L, 3)
    l1_full = xyz_all - ca_all[:, None, :]
    l1_blk = xyz_blk - ca_blk[:, None, :]

    avs = [jnp.sum(gate0[:, :, a:a + 1] * l1_full[:, a, :][None, :, :], axis=1)
           + w['bg'][0:1, a:a + 1] * _dot(mask_l, l1_full[:, a, :])
           for a in range(3)]
    agg_vec = jnp.stack(avs, axis=1) / float(TOP_K) + l1_blk

    vm = w['vecmix']
    Toff = (vm[0:1, 0:1] * agg_vec[:, 0, :] + vm[0:1, 1:2] * agg_vec[:, 1, :]
            + vm[0:1, 2:3] * agg_vec[:, 2, :]) / 10.0
    Roff = (vm[1:2, 0:1] * agg_vec[:, 0, :] + vm[1:2, 1:2] * agg_vec[:, 1, :]
            + vm[1:2, 2:3] * agg_vec[:, 2, :]) / 100.0
    Qn = jnp.sqrt(1.0 + jnp.sum(Roff * Roff, -1, keepdims=True))
    qA = 1.0 / Qn
    qB = Roff[:, 0:1] / Qn
    qC = Roff[:, 1:2] / Qn
    qD = Roff[:, 2:3] / Qn
    r00 = qA * qA + qB * qB - qC * qC - qD * qD
    r01 = 2 * qB * qC - 2 * qA * qD
    r02 = 2 * qB * qD + 2 * qA * qC
    r10 = 2 * qB * qC + 2 * qA * qD
    r11 = qA * qA - qB * qB + qC * qC - qD * qD
    r12 = 2 * qC * qD - 2 * qA * qB
    r20 = 2 * qB * qD - 2 * qA * qC
    r21 = 2 * qC * qD + 2 * qA * qB
    r22 = qA * qA - qB * qB - qC * qC + qD * qD
    v = l1_blk
    vx = v[:, :, 0]
    vy = v[:, :, 1]
    vz = v[:, :, 2]
    xn0 = r00 * vx + r01 * vy + r02 * vz
    xn1 = r10 * vx + r11 * vy + r12 * vz
    xn2 = r20 * vx + r21 * vy + r22 * vz
    xyz_new = (jnp.stack([xn0, xn1, xn2], axis=-1)
               + ca_blk[:, None, :] + Toff[:, None, :])
    oxyz_ref[:] = xyz_new
    ostate_ref[:] = state_out

    msa_blk = msa_ref[pl.ds(r0, BI), :]
    s = _ln(msa_blk, w['g_s0'], w['b_s0'])
    st2 = _ln(state_out, w['g_si'], w['b_si'])
    si = _dot(s, w['Ws0']) + w['bs0'] + _dot(st2, w['Wsi']) + w['bsi']
    si = si + _dot(jax.nn.relu(_dot(jax.nn.relu(si), w['W1']) + w['b1']),
                   w['W2']) + w['b2']
    si = si + _dot(jax.nn.relu(_dot(jax.nn.relu(si), w['W3']) + w['b3']),
                   w['W4']) + w['b4']
    oalpha_ref[:] = _dot(jax.nn.relu(si), w['Wo']) + w['bo']


@functools.partial(jax.jit, static_argnames=())
def _run(msa0, state0, xyz0, caT, idxc, idxr, pair2, w):
    full = lambda a: pl.BlockSpec(a.shape, lambda i, nd=a.ndim: (0,) * nd)
    wspecs = jax.tree.map(full, w)
    grid = (L // BI,)
    out = pl.pallas_call(
        _body,
        grid=grid,
        in_specs=[full(msa0), full(state0), full(xyz0), full(caT),
                  full(idxc), full(idxr),
                  pl.BlockSpec((BI, L, 128), lambda i: (i, 0, 0)),
                  wspecs],
        out_specs=[pl.BlockSpec((BI, 3, 3), lambda i: (i, 0, 0)),
                   pl.BlockSpec((BI, 32), lambda i: (i, 0)),
                   pl.BlockSpec((BI, 20), lambda i: (i, 0))],
        out_shape=[jax.ShapeDtypeStruct((L, 3, 3), jnp.float32),
                   jax.ShapeDtypeStruct((L, 32), jnp.float32),
                   jax.ShapeDtypeStruct((L, 20), jnp.float32)],
        compiler_params=pltpu.CompilerParams(
            dimension_semantics=("arbitrary",),
            vmem_limit_bytes=120 * 2**20),
    )(msa0, state0, xyz0, caT, idxc, idxr, pair2, w)
    return out


def kernel(msa, pair, xyz, state, idx, params):
    p = params
    msa0 = msa[0, 0]
    pair2 = pair[0]
    xyz0 = xyz[0]
    state0 = state[0]
    caT = xyz0[:, 1, :].T
    idxc = idx.astype(jnp.float32).reshape(L, 1)
    idxr = idx.astype(jnp.float32).reshape(1, L)

    row = lambda a: a.reshape(1, -1)
    W1 = p['embed_e1']['W']
    g_p, b_p = p['norm_pair']['g'], p['norm_pair']['b']
    W1g = W1.T * g_p[:, None]
    g1, b1n = p['norm_edge1']['g'], p['norm_edge1']['b']
    W2T = p['embed_e2']['W'].T
    W2cat = jnp.concatenate([W2T[:32] * g1[:, None], W2T[32:]], axis=0)
    g2, b2n = p['norm_edge2']['g'], p['norm_edge2']['b']
    WmE_T = p['se3_msg']['W'][:, 64:].T
    w = {
        'g_msa': row(p['norm_msa']['g']), 'b_msa': row(p['norm_msa']['b']),
        'g_state': row(p['norm_state']['g']), 'b_state': row(p['norm_state']['b']),
        'Wx_m': p['embed_x']['W'][:, :256].T, 'Wx_s': p['embed_x']['W'][:, 256:].T,
        'bx': row(p['embed_x']['b']),
        'g_node': row(p['norm_node']['g']), 'b_node': row(p['norm_node']['b']),
        'W1g': W1g,
        'w1sum': jnp.sum(W1g, axis=0, keepdims=True),
        'be1tot': row(b_p @ W1.T + p['embed_e1']['b']),
        'W2cat': W2cat,
        'be2tot': row(b1n @ W2T[:32] + p['embed_e2']['b']),
        'Wm_self': p['se3_msg']['W'][:, :32].T,
        'Wm_nbr': p['se3_msg']['W'][:, 32:64].T,
        'Wm_edge': WmE_T * g2[:, None],
        'bm': row(p['se3_msg']['b'] + b2n @ WmE_T),
        'Wl0_node': p['se3_l0']['W'][:, :32].T,
        'Wl0_agg': p['se3_l0']['W'][:, 32:].T,
        'bl0': row(p['se3_l0']['b']),
        'Wg': p['se3_gate']['W'].T, 'bg': row(p['se3_gate']['b']),
        'vecmix': p['se3_vecmix'],
        'g_s0': row(p['sc_norm_s0']['g']), 'b_s0': row(p['sc_norm_s0']['b']),
        'g_si': row(p['sc_norm_si']['g']), 'b_si': row(p['sc_norm_si']['b']),
        'Ws0': p['sc_s0']['W'].T, 'bs0': row(p['sc_s0']['b']),
        'Wsi': p['sc_si']['W'].T, 'bsi': row(p['sc_si']['b']),
        'W1': p['sc_l1']['W'].T, 'b1': row(p['sc_l1']['b']),
        'W2': p['sc_l2']['W'].T, 'b2': row(p['sc_l2']['b']),
        'W3': p['sc_l3']['W'].T, 'b3': row(p['sc_l3']['b']),
        'W4': p['sc_l4']['W'].T, 'b4': row(p['sc_l4']['b']),
        'Wo': p['sc_out']['W'].T, 'bo': row(p['sc_out']['b']),
    }
    xyz_new, state_out, alpha = _run(msa0, state0, xyz0, caT, idxc, idxr,
                                     pair2, w)
    return (xyz_new[None], state_out[None], alpha.reshape(1, L, 10, 2))
